# baseline (speedup 1.0000x reference)
import functools
import math

import jax
import jax.numpy as jnp
import numpy as np
from jax import lax
from jax.experimental import pallas as pl
from jax.experimental.pallas import tpu as pltpu
from jax.experimental.pallas import tpu_sc as plsc

_MAX_LEN = 288
_EMBED_DIM = 128


def _sin_enc(max_len, dim, period):
    position = np.arange(max_len, dtype=np.float32)[:, None]
    div_term = np.exp(np.arange(0, dim, 2, dtype=np.float32) * -(math.log(period) / dim))
    pe = np.zeros((max_len, dim), dtype=np.float32)
    pe[:, 0::2] = np.sin(position * div_term)
    pe[:, 1::2] = np.cos(position * div_term)
    return pe


def _build_pe_np():
    pe_standard = _sin_enc(_MAX_LEN, _EMBED_DIM // 2, 10000.0)
    pe_daily = _sin_enc(_MAX_LEN, _EMBED_DIM // 4, 288.0)
    pe_weekly = _sin_enc(_MAX_LEN, _EMBED_DIM // 4, 288.0 * 7)
    return np.concatenate([pe_standard, pe_daily, pe_weekly], axis=-1)


_PE = _build_pe_np()

_ROW = _MAX_LEN * _EMBED_DIM
_NW = 32
_LANES = 16


def _sc_add(x2, pe_flat):
    rows = x2.shape[0]
    rows_per_w = rows // _NW
    nvec = _ROW // _LANES

    mesh = plsc.VectorSubcoreMesh(core_axis_name="c", subcore_axis_name="s")

    @functools.partial(
        pl.kernel,
        mesh=mesh,
        out_type=jax.ShapeDtypeStruct((rows, _ROW), jnp.float32),
        scratch_types=[
            pltpu.VMEM((_ROW,), jnp.float32),
            pltpu.VMEM((_ROW,), jnp.float32),
        ],
    )
    def k(x_hbm, pe_hbm, out_hbm, buf, pe_v):
        wid = lax.axis_index("s") * 2 + lax.axis_index("c")
        pltpu.sync_copy(pe_hbm, pe_v)
        base = wid * rows_per_w

        def row_body(r, _):
            row = base + r
            pltpu.sync_copy(x_hbm.at[row], buf)

            def add_body(i, _):
                sl = pl.ds(i * _LANES, _LANES)
                buf[sl] = buf[sl] + pe_v[sl]
                return 0

            lax.fori_loop(0, nvec, add_body, 0, unroll=8)
            pltpu.sync_copy(buf, out_hbm.at[row])
            return 0

        lax.fori_loop(0, rows_per_w, row_body, 0)

    return k(x2, pe_flat)


def kernel(x):
    B, N, T, D = x.shape
    rows = B * N
    pe_flat = jnp.asarray(_PE[:T].reshape(-1))
    x2 = x.reshape(rows, T * D)
    out = _sc_add(x2, pe_flat)
    return out.reshape(B, N, T, D)


# speedup vs baseline: 1.2562x; 1.2562x over previous
import functools
import math

import jax
import jax.numpy as jnp
import numpy as np
from jax import lax
from jax.experimental import pallas as pl
from jax.experimental.pallas import tpu as pltpu
from jax.experimental.pallas import tpu_sc as plsc

_MAX_LEN = 288
_EMBED_DIM = 128


def _sin_enc(max_len, dim, period):
    position = np.arange(max_len, dtype=np.float32)[:, None]
    div_term = np.exp(np.arange(0, dim, 2, dtype=np.float32) * -(math.log(period) / dim))
    pe = np.zeros((max_len, dim), dtype=np.float32)
    pe[:, 0::2] = np.sin(position * div_term)
    pe[:, 1::2] = np.cos(position * div_term)
    return pe


def _build_pe_np():
    pe_standard = _sin_enc(_MAX_LEN, _EMBED_DIM // 2, 10000.0)
    pe_daily = _sin_enc(_MAX_LEN, _EMBED_DIM // 4, 288.0)
    pe_weekly = _sin_enc(_MAX_LEN, _EMBED_DIM // 4, 288.0 * 7)
    return np.concatenate([pe_standard, pe_daily, pe_weekly], axis=-1)


_PE = _build_pe_np()

_ROW = _MAX_LEN * _EMBED_DIM
_HALF = _ROW // 2
_NW = 32
_LANES = 16
_NVEC = _HALF // _LANES


def _sc_add(x2, pe_flat):
    units = x2.shape[0]
    units_per_w = units // _NW

    mesh = plsc.VectorSubcoreMesh(core_axis_name="c", subcore_axis_name="s")

    @functools.partial(
        pl.kernel,
        mesh=mesh,
        out_type=jax.ShapeDtypeStruct((units, _HALF), jnp.float32),
        scratch_types=[
            pltpu.VMEM((_HALF,), jnp.float32),
            pltpu.VMEM((_HALF,), jnp.float32),
            pltpu.VMEM((_HALF,), jnp.float32),
            pltpu.VMEM((_HALF,), jnp.float32),
            pltpu.VMEM((_ROW,), jnp.float32),
            pltpu.SemaphoreType.DMA,
            pltpu.SemaphoreType.DMA,
            pltpu.SemaphoreType.DMA,
            pltpu.SemaphoreType.DMA,
        ],
    )
    def k(x_hbm, pe_hbm, out_hbm, in0, in1, o0, o1, pe_v, si0, si1, so0, so1):
        wid = lax.axis_index("s") * 2 + lax.axis_index("c")
        pltpu.sync_copy(pe_hbm, pe_v)
        base = wid * units_per_w

        pltpu.async_copy(x_hbm.at[base], in0, si0)
        pltpu.async_copy(x_hbm.at[base + 1], in1, si1)

        def do(u, slot_in, slot_out, s_in, s_out):
            gu = base + u
            peoff = (u % 2) * _HALF

            pltpu.make_async_copy(x_hbm.at[gu], slot_in, s_in).wait()

            @pl.when(u >= 2)
            def _():
                pltpu.make_async_copy(slot_out, out_hbm.at[gu], s_out).wait()

            def add_body(i, _):
                sl = pl.ds(i * _LANES, _LANES)
                slp = pl.ds(peoff + i * _LANES, _LANES)
                slot_out[sl] = slot_in[sl] + pe_v[slp]
                return 0

            lax.fori_loop(0, _NVEC, add_body, 0, unroll=8)

            pltpu.async_copy(slot_out, out_hbm.at[gu], s_out)

            @pl.when(u + 2 < units_per_w)
            def _():
                pltpu.async_copy(x_hbm.at[gu + 2], slot_in, s_in)

        def body(u, _):
            lax.cond(
                u % 2 == 0,
                lambda: do(u, in0, o0, si0, so0),
                lambda: do(u, in1, o1, si1, so1),
            )
            return 0

        lax.fori_loop(0, units_per_w, body, 0)

        pltpu.make_async_copy(o0, out_hbm.at[base], so0).wait()
        pltpu.make_async_copy(o1, out_hbm.at[base + 1], so1).wait()

    return k(x2, pe_flat)


def kernel(x):
    B, N, T, D = x.shape
    pe_flat = jnp.asarray(_PE[:T].reshape(-1))
    x2 = x.reshape(B * N * 2, _HALF)
    out = _sc_add(x2, pe_flat)
    return out.reshape(B, N, T, D)


# speedup vs baseline: 1.6620x; 1.3230x over previous
import functools
import math

import jax
import jax.numpy as jnp
import numpy as np
from jax import lax
from jax.experimental import pallas as pl
from jax.experimental.pallas import tpu as pltpu
from jax.experimental.pallas import tpu_sc as plsc

_MAX_LEN = 288
_EMBED_DIM = 128


def _sin_enc(max_len, dim, period):
    position = np.arange(max_len, dtype=np.float32)[:, None]
    div_term = np.exp(np.arange(0, dim, 2, dtype=np.float32) * -(math.log(period) / dim))
    pe = np.zeros((max_len, dim), dtype=np.float32)
    pe[:, 0::2] = np.sin(position * div_term)
    pe[:, 1::2] = np.cos(position * div_term)
    return pe


def _build_pe_np():
    pe_standard = _sin_enc(_MAX_LEN, _EMBED_DIM // 2, 10000.0)
    pe_daily = _sin_enc(_MAX_LEN, _EMBED_DIM // 4, 288.0)
    pe_weekly = _sin_enc(_MAX_LEN, _EMBED_DIM // 4, 288.0 * 7)
    return np.concatenate([pe_standard, pe_daily, pe_weekly], axis=-1)


_PE = _build_pe_np()

_ROW = _MAX_LEN * _EMBED_DIM
_HALF = _ROW // 2
_NW = 32
_LANES = 16
_NVEC = _HALF // _LANES


def _sc_add(x2, pe_flat):
    units = x2.shape[0]
    units_per_w = units // _NW

    mesh = plsc.VectorSubcoreMesh(core_axis_name="c", subcore_axis_name="s")

    @functools.partial(
        pl.kernel,
        mesh=mesh,
        out_type=jax.ShapeDtypeStruct((units, _HALF), jnp.float32),
        scratch_types=[
            pltpu.VMEM((_HALF,), jnp.float32),
            pltpu.VMEM((_HALF,), jnp.float32),
            pltpu.VMEM((_HALF,), jnp.float32),
            pltpu.VMEM((_HALF,), jnp.float32),
            pltpu.VMEM((_ROW,), jnp.float32),
            pltpu.SemaphoreType.DMA,
            pltpu.SemaphoreType.DMA,
            pltpu.SemaphoreType.DMA,
            pltpu.SemaphoreType.DMA,
        ],
    )
    def k(x_hbm, pe_hbm, out_hbm, in0, in1, o0, o1, pe_v, si0, si1, so0, so1):
        wid = lax.axis_index("s") * 2 + lax.axis_index("c")
        pltpu.sync_copy(pe_hbm, pe_v)
        base = wid * units_per_w

        pltpu.async_copy(x_hbm.at[base], in0, si0)
        pltpu.async_copy(x_hbm.at[base + 1], in1, si1)

        def do(u, slot_in, slot_out, s_in, s_out):
            gu = base + u
            peoff = (u % 2) * _HALF

            pltpu.make_async_copy(x_hbm.at[gu], slot_in, s_in).wait()

            @pl.when(u >= 2)
            def _():
                pltpu.make_async_copy(slot_out, out_hbm.at[gu], s_out).wait()

            @plsc.parallel_loop(0, _HALF, step=_LANES, unroll=8)
            def add_body(i):
                sl = pl.ds(i, _LANES)
                slp = pl.ds(peoff + i, _LANES)
                slot_out[sl] = slot_in[sl] + pe_v[slp]

            pltpu.async_copy(slot_out, out_hbm.at[gu], s_out)

            @pl.when(u + 2 < units_per_w)
            def _():
                pltpu.async_copy(x_hbm.at[gu + 2], slot_in, s_in)

        def body(u, _):
            lax.cond(
                u % 2 == 0,
                lambda: do(u, in0, o0, si0, so0),
                lambda: do(u, in1, o1, si1, so1),
            )
            return 0

        lax.fori_loop(0, units_per_w, body, 0)

        pltpu.make_async_copy(o0, out_hbm.at[base], so0).wait()
        pltpu.make_async_copy(o1, out_hbm.at[base + 1], so1).wait()

    return k(x2, pe_flat)


def kernel(x):
    B, N, T, D = x.shape
    pe_flat = jnp.asarray(_PE[:T].reshape(-1))
    x2 = x.reshape(B * N * 2, _HALF)
    out = _sc_add(x2, pe_flat)
    return out.reshape(B, N, T, D)


# speedup vs baseline: 5.6504x; 3.3999x over previous
import functools
import math

import jax
import jax.numpy as jnp
import numpy as np
from jax import lax
from jax.experimental import pallas as pl
from jax.experimental.pallas import tpu as pltpu
from jax.experimental.pallas import tpu_sc as plsc

_MAX_LEN = 288
_EMBED_DIM = 128


def _sin_enc(max_len, dim, period):
    position = np.arange(max_len, dtype=np.float32)[:, None]
    div_term = np.exp(np.arange(0, dim, 2, dtype=np.float32) * -(math.log(period) / dim))
    pe = np.zeros((max_len, dim), dtype=np.float32)
    pe[:, 0::2] = np.sin(position * div_term)
    pe[:, 1::2] = np.cos(position * div_term)
    return pe


def _build_pe_np():
    pe_standard = _sin_enc(_MAX_LEN, _EMBED_DIM // 2, 10000.0)
    pe_daily = _sin_enc(_MAX_LEN, _EMBED_DIM // 4, 288.0)
    pe_weekly = _sin_enc(_MAX_LEN, _EMBED_DIM // 4, 288.0 * 7)
    return np.concatenate([pe_standard, pe_daily, pe_weekly], axis=-1)


_PE = _build_pe_np()

_HT = _MAX_LEN // 2
_NW = 32
_LANES = 16
_D_SL = _EMBED_DIM // _LANES


def _sc_add(x3, pe):
    units = x3.shape[0]
    units_per_w = units // _NW

    mesh = plsc.VectorSubcoreMesh(core_axis_name="c", subcore_axis_name="s")

    @functools.partial(
        pl.kernel,
        mesh=mesh,
        out_type=jax.ShapeDtypeStruct((units, _HT, _EMBED_DIM), jnp.float32),
        scratch_types=[
            pltpu.VMEM((_HT, _EMBED_DIM), jnp.float32),
            pltpu.VMEM((_HT, _EMBED_DIM), jnp.float32),
            pltpu.VMEM((_HT, _EMBED_DIM), jnp.float32),
            pltpu.VMEM((_HT, _EMBED_DIM), jnp.float32),
            pltpu.VMEM((_MAX_LEN, _EMBED_DIM), jnp.float32),
            pltpu.SemaphoreType.DMA,
            pltpu.SemaphoreType.DMA,
            pltpu.SemaphoreType.DMA,
            pltpu.SemaphoreType.DMA,
        ],
    )
    def k(x_hbm, pe_hbm, out_hbm, in0, in1, o0, o1, pe_v, si0, si1, so0, so1):
        wid = lax.axis_index("s") * 2 + lax.axis_index("c")
        pltpu.sync_copy(pe_hbm, pe_v)
        base = wid * units_per_w

        pltpu.async_copy(x_hbm.at[base], in0, si0)
        pltpu.async_copy(x_hbm.at[base + 1], in1, si1)

        def do(u, slot_in, slot_out, s_in, s_out):
            gu = base + u
            peoff = (u % 2) * _HT

            pltpu.make_async_copy(x_hbm.at[gu], slot_in, s_in).wait()

            @pl.when(u >= 2)
            def _():
                pltpu.make_async_copy(slot_out, out_hbm.at[gu], s_out).wait()

            @plsc.parallel_loop(0, _HT, step=1, unroll=2)
            def add_body(r):
                pr = peoff + r
                for j in range(_D_SL):
                    sl = pl.ds(j * _LANES, _LANES)
                    slot_out[r, sl] = slot_in[r, sl] + pe_v[pr, sl]

            pltpu.async_copy(slot_out, out_hbm.at[gu], s_out)

            @pl.when(u + 2 < units_per_w)
            def _():
                pltpu.async_copy(x_hbm.at[gu + 2], slot_in, s_in)

        def body(u, _):
            lax.cond(
                u % 2 == 0,
                lambda: do(u, in0, o0, si0, so0),
                lambda: do(u, in1, o1, si1, so1),
            )
            return 0

        lax.fori_loop(0, units_per_w, body, 0)

        pltpu.make_async_copy(o0, out_hbm.at[base], so0).wait()
        pltpu.make_async_copy(o1, out_hbm.at[base + 1], so1).wait()

    return k(x3, pe)


def kernel(x):
    B, N, T, D = x.shape
    pe = jnp.asarray(_PE[:T])
    x3 = x.reshape(B * N * 2, _HT, D)
    out = _sc_add(x3, pe)
    return out.reshape(B, N, T, D)


# speedup vs baseline: 5.7851x; 1.0238x over previous
import functools
import math

import jax
import jax.numpy as jnp
import numpy as np
from jax import lax
from jax.experimental import pallas as pl
from jax.experimental.pallas import tpu as pltpu
from jax.experimental.pallas import tpu_sc as plsc

_MAX_LEN = 288
_EMBED_DIM = 128


def _sin_enc(max_len, dim, period):
    position = np.arange(max_len, dtype=np.float32)[:, None]
    div_term = np.exp(np.arange(0, dim, 2, dtype=np.float32) * -(math.log(period) / dim))
    pe = np.zeros((max_len, dim), dtype=np.float32)
    pe[:, 0::2] = np.sin(position * div_term)
    pe[:, 1::2] = np.cos(position * div_term)
    return pe


def _build_pe_np():
    pe_standard = _sin_enc(_MAX_LEN, _EMBED_DIM // 2, 10000.0)
    pe_daily = _sin_enc(_MAX_LEN, _EMBED_DIM // 4, 288.0)
    pe_weekly = _sin_enc(_MAX_LEN, _EMBED_DIM // 4, 288.0 * 7)
    return np.concatenate([pe_standard, pe_daily, pe_weekly], axis=-1)


_PE = _build_pe_np()

_HT = _MAX_LEN // 2
_NW = 32
_LANES = 16
_D_SL = _EMBED_DIM // _LANES


def _sc_add(x3, pe):
    units = x3.shape[0]
    units_per_w = units // _NW

    mesh = plsc.VectorSubcoreMesh(core_axis_name="c", subcore_axis_name="s")

    depth = 4

    @functools.partial(
        pl.kernel,
        mesh=mesh,
        out_type=jax.ShapeDtypeStruct((units, _HT, _EMBED_DIM), jnp.float32),
        scratch_types=(
            [pltpu.VMEM((_HT, _EMBED_DIM), jnp.float32) for _ in range(depth)]
            + [pltpu.VMEM((_MAX_LEN, _EMBED_DIM), jnp.float32)]
            + [pltpu.SemaphoreType.DMA for _ in range(2 * depth)]
        ),
    )
    def k(x_hbm, pe_hbm, out_hbm, b0, b1, b2, b3, pe_v, *sems):
        bufs = (b0, b1, b2, b3)
        sin = sems[:depth]
        sout = sems[depth:]
        wid = lax.axis_index("s") * 2 + lax.axis_index("c")
        pltpu.sync_copy(pe_hbm, pe_v)
        base = wid * units_per_w

        for p in range(depth):
            pltpu.async_copy(x_hbm.at[base + p], bufs[p], sin[p])

        def do(u, p):
            gu = base + u
            buf, s_in, s_out = bufs[p], sin[p], sout[p]
            peoff = (u % 2) * _HT

            pltpu.make_async_copy(x_hbm.at[gu], buf, s_in).wait()

            @plsc.parallel_loop(0, _HT, step=1, unroll=2)
            def add_body(r):
                pr = peoff + r
                for j in range(_D_SL):
                    sl = pl.ds(j * _LANES, _LANES)
                    plsc.addupdate(buf.at[r, sl], pe_v[pr, sl])

            pltpu.async_copy(buf, out_hbm.at[gu], s_out)

            t = (p + depth - 1) % depth
            nxt = u + depth - 1

            @pl.when(jnp.logical_and(u >= 1, nxt < units_per_w))
            def _():
                pltpu.make_async_copy(bufs[t], out_hbm.at[gu], sout[t]).wait()
                pltpu.async_copy(x_hbm.at[base + nxt], bufs[t], sin[t])

        def body(u, _):
            lax.switch(u % depth, [functools.partial(do, u, p) for p in range(depth)])
            return 0

        lax.fori_loop(0, units_per_w, body, 0)

        for p in range(depth):
            pltpu.make_async_copy(bufs[p], out_hbm.at[base], sout[p]).wait()

    return k(x3, pe)


def kernel(x):
    B, N, T, D = x.shape
    pe = jnp.asarray(_PE[:T])
    x3 = x.reshape(B * N * 2, _HT, D)
    out = _sc_add(x3, pe)
    return out.reshape(B, N, T, D)


# speedup vs baseline: 5.7942x; 1.0016x over previous
import functools
import math

import jax
import jax.numpy as jnp
import numpy as np
from jax import lax
from jax.experimental import pallas as pl
from jax.experimental.pallas import tpu as pltpu
from jax.experimental.pallas import tpu_sc as plsc

_MAX_LEN = 288
_EMBED_DIM = 128


def _sin_enc(max_len, dim, period):
    position = np.arange(max_len, dtype=np.float32)[:, None]
    div_term = np.exp(np.arange(0, dim, 2, dtype=np.float32) * -(math.log(period) / dim))
    pe = np.zeros((max_len, dim), dtype=np.float32)
    pe[:, 0::2] = np.sin(position * div_term)
    pe[:, 1::2] = np.cos(position * div_term)
    return pe


def _build_pe_np():
    pe_standard = _sin_enc(_MAX_LEN, _EMBED_DIM // 2, 10000.0)
    pe_daily = _sin_enc(_MAX_LEN, _EMBED_DIM // 4, 288.0)
    pe_weekly = _sin_enc(_MAX_LEN, _EMBED_DIM // 4, 288.0 * 7)
    return np.concatenate([pe_standard, pe_daily, pe_weekly], axis=-1)


_PE = _build_pe_np()

_HT = _MAX_LEN // 4
_SLABS = _MAX_LEN // _HT
_NW = 32
_LANES = 16
_D_SL = _EMBED_DIM // _LANES


def _sc_add(x3, pe):
    units = x3.shape[0]
    units_per_w = units // _NW

    mesh = plsc.VectorSubcoreMesh(core_axis_name="c", subcore_axis_name="s")

    depth = 8

    @functools.partial(
        pl.kernel,
        mesh=mesh,
        out_type=jax.ShapeDtypeStruct((units, _HT, _EMBED_DIM), jnp.float32),
        scratch_types=(
            [pltpu.VMEM((_HT, _EMBED_DIM), jnp.float32) for _ in range(depth)]
            + [pltpu.VMEM((_MAX_LEN, _EMBED_DIM), jnp.float32)]
            + [pltpu.SemaphoreType.DMA for _ in range(2 * depth)]
        ),
    )
    def k(x_hbm, pe_hbm, out_hbm, b0, b1, b2, b3, b4, b5, b6, b7, pe_v, *sems):
        bufs = (b0, b1, b2, b3, b4, b5, b6, b7)
        sin = sems[:depth]
        sout = sems[depth:]
        wid = lax.axis_index("s") * 2 + lax.axis_index("c")
        pltpu.sync_copy(pe_hbm, pe_v)
        base = wid * units_per_w

        for p in range(depth):
            pltpu.async_copy(x_hbm.at[base + p], bufs[p], sin[p])

        def do(u, p):
            gu = base + u
            buf, s_in, s_out = bufs[p], sin[p], sout[p]
            peoff = (u % _SLABS) * _HT

            pltpu.make_async_copy(x_hbm.at[gu], buf, s_in).wait()

            @plsc.parallel_loop(0, _HT, step=1, unroll=2)
            def add_body(r):
                pr = peoff + r
                for j in range(_D_SL):
                    sl = pl.ds(j * _LANES, _LANES)
                    plsc.addupdate(buf.at[r, sl], pe_v[pr, sl])

            pltpu.async_copy(buf, out_hbm.at[gu], s_out)

            t = (p + depth - 1) % depth
            nxt = u + depth - 1

            @pl.when(jnp.logical_and(u >= 1, nxt < units_per_w))
            def _():
                pltpu.make_async_copy(bufs[t], out_hbm.at[gu], sout[t]).wait()
                pltpu.async_copy(x_hbm.at[base + nxt], bufs[t], sin[t])

        def body(u, _):
            lax.switch(u % depth, [functools.partial(do, u, p) for p in range(depth)])
            return 0

        lax.fori_loop(0, units_per_w, body, 0)

        for p in range(depth):
            pltpu.make_async_copy(bufs[p], out_hbm.at[base], sout[p]).wait()

    return k(x3, pe)


def kernel(x):
    B, N, T, D = x.shape
    pe = jnp.asarray(_PE[:T])
    x3 = x.reshape(B * N * _SLABS, _HT, D)
    out = _sc_add(x3, pe)
    return out.reshape(B, N, T, D)


# speedup vs baseline: 6.1967x; 1.0695x over previous
import functools
import math

import jax
import jax.numpy as jnp
import numpy as np
from jax import lax
from jax.experimental import pallas as pl
from jax.experimental.pallas import tpu as pltpu
from jax.experimental.pallas import tpu_sc as plsc

_MAX_LEN = 288
_EMBED_DIM = 128


def _sin_enc(max_len, dim, period):
    position = np.arange(max_len, dtype=np.float32)[:, None]
    div_term = np.exp(np.arange(0, dim, 2, dtype=np.float32) * -(math.log(period) / dim))
    pe = np.zeros((max_len, dim), dtype=np.float32)
    pe[:, 0::2] = np.sin(position * div_term)
    pe[:, 1::2] = np.cos(position * div_term)
    return pe


def _build_pe_np():
    pe_standard = _sin_enc(_MAX_LEN, _EMBED_DIM // 2, 10000.0)
    pe_daily = _sin_enc(_MAX_LEN, _EMBED_DIM // 4, 288.0)
    pe_weekly = _sin_enc(_MAX_LEN, _EMBED_DIM // 4, 288.0 * 7)
    return np.concatenate([pe_standard, pe_daily, pe_weekly], axis=-1)


_PE = _build_pe_np()

_HT = _MAX_LEN // 4
_SLABS = _MAX_LEN // _HT
_NW = 32
_LANES = 16
_D_SL = _EMBED_DIM // _LANES

_SC_FRAC_NUM, _SC_FRAC_DEN = 1, 2
_TC_CHUNK = 64


def _sc_add(x3, pe, sc_units):
    units = x3.shape[0]
    units_per_w = sc_units // _NW

    mesh = plsc.VectorSubcoreMesh(core_axis_name="c", subcore_axis_name="s")

    depth = 8

    @functools.partial(
        pl.kernel,
        mesh=mesh,
        out_type=jax.ShapeDtypeStruct((units, _HT, _EMBED_DIM), jnp.float32),
        scratch_types=(
            [pltpu.VMEM((_HT, _EMBED_DIM), jnp.float32) for _ in range(depth)]
            + [pltpu.VMEM((_MAX_LEN, _EMBED_DIM), jnp.float32)]
            + [pltpu.SemaphoreType.DMA for _ in range(2 * depth)]
        ),
    )
    def k(x_hbm, pe_hbm, out_hbm, b0, b1, b2, b3, b4, b5, b6, b7, pe_v, *sems):
        bufs = (b0, b1, b2, b3, b4, b5, b6, b7)
        sin = sems[:depth]
        sout = sems[depth:]
        wid = lax.axis_index("s") * 2 + lax.axis_index("c")
        pltpu.sync_copy(pe_hbm, pe_v)
        base = wid * units_per_w

        for p in range(depth):
            pltpu.async_copy(x_hbm.at[base + p], bufs[p], sin[p])

        def do(u, p):
            gu = base + u
            buf, s_in, s_out = bufs[p], sin[p], sout[p]
            peoff = (u % _SLABS) * _HT

            pltpu.make_async_copy(x_hbm.at[gu], buf, s_in).wait()

            @plsc.parallel_loop(0, _HT, step=1, unroll=2)
            def add_body(r):
                pr = peoff + r
                for j in range(_D_SL):
                    sl = pl.ds(j * _LANES, _LANES)
                    plsc.addupdate(buf.at[r, sl], pe_v[pr, sl])

            pltpu.async_copy(buf, out_hbm.at[gu], s_out)

            t = (p + depth - 1) % depth
            nxt = u + depth - 1

            @pl.when(jnp.logical_and(u >= 1, nxt < units_per_w))
            def _():
                pltpu.make_async_copy(bufs[t], out_hbm.at[gu], sout[t]).wait()
                pltpu.async_copy(x_hbm.at[base + nxt], bufs[t], sin[t])

        def body(u, _):
            lax.switch(u % depth, [functools.partial(do, u, p) for p in range(depth)])
            return 0

        lax.fori_loop(0, units_per_w, body, 0)

        for p in range(depth):
            pltpu.make_async_copy(bufs[p], out_hbm.at[base], sout[p]).wait()

    return k(x3, pe)


def _tc_body(x_ref, pe_ref, alias_ref, o_ref):
    del alias_ref
    o_ref[...] = x_ref[...] + pe_ref[...]


def _tc_fill(x2, pe, sc2, start_row):
    rows = x2.shape[0]
    grid = (rows - start_row) // _TC_CHUNK
    off = start_row // _TC_CHUNK
    return pl.pallas_call(
        _tc_body,
        grid=(grid,),
        in_specs=[
            pl.BlockSpec((_TC_CHUNK, _MAX_LEN, _EMBED_DIM), lambda i: (off + i, 0, 0)),
            pl.BlockSpec((1, _MAX_LEN, _EMBED_DIM), lambda i: (0, 0, 0)),
            pl.BlockSpec(memory_space=pl.ANY),
        ],
        out_specs=pl.BlockSpec((_TC_CHUNK, _MAX_LEN, _EMBED_DIM), lambda i: (off + i, 0, 0)),
        out_shape=jax.ShapeDtypeStruct((rows, _MAX_LEN, _EMBED_DIM), jnp.float32),
        input_output_aliases={2: 0},
    )(x2, pe[None], sc2)


def kernel(x):
    B, N, T, D = x.shape
    pe = jnp.asarray(_PE[:T])
    rows = B * N
    sc_rows = rows * _SC_FRAC_NUM // _SC_FRAC_DEN
    sc_units = sc_rows * _SLABS

    x3 = x.reshape(rows * _SLABS, _HT, D)
    sc_out = _sc_add(x3, pe, sc_units)

    x2 = x.reshape(rows, T, D)
    sc2 = sc_out.reshape(rows, T, D)
    out = _tc_fill(x2, pe, sc2, sc_rows)
    return out.reshape(B, N, T, D)
